# baseline (speedup 1.0000x reference)
import jax
import jax.numpy as jnp
from jax.experimental import pallas as pl
from jax.experimental.pallas import tpu as pltpu

_NQ = 4


def _router_body(x0, x1, x2, x3, m_ref, w_ref, probs_ref, logits_ref):
    w = w_ref[...]
    bq = x0.shape[0]
    for q, xq in enumerate((x0, x1, x2, x3)):
        logits = jnp.dot(xq[...], w, preferred_element_type=jnp.float32)
        mx = jnp.max(logits, axis=-1, keepdims=True)
        e = jnp.exp(logits - mx)
        s = jnp.sum(e, axis=-1, keepdims=True)
        sl = pl.ds(q * bq, bq)
        probs_ref[sl, :] = (e / s) * m_ref[sl, :]
        logits_ref[sl, :] = logits


def kernel(inputs, padding_mask, w, num_experts):
    T, D = inputs.shape
    E = w.shape[1]
    BT = 4096
    BQ = BT // _NQ
    x_specs = [
        pl.BlockSpec((BQ, D), lambda i, q=q: (_NQ * i + q, 0)) for q in range(_NQ)
    ]
    probs, logits = pl.pallas_call(
        _router_body,
        grid=(T // BT,),
        in_specs=x_specs
        + [
            pl.BlockSpec((BT, 1), lambda i: (i, 0)),
            pl.BlockSpec((D, E), lambda i: (0, 0)),
        ],
        out_specs=[
            pl.BlockSpec((BT, E), lambda i: (i, 0)),
            pl.BlockSpec((BT, E), lambda i: (i, 0)),
        ],
        out_shape=[
            jax.ShapeDtypeStruct((T, E), jnp.float32),
            jax.ShapeDtypeStruct((T, E), jnp.float32),
        ],
        compiler_params=pltpu.CompilerParams(
            dimension_semantics=("arbitrary",),
        ),
    )(inputs, inputs, inputs, inputs, padding_mask, w)
    return (probs, logits)
